# baseline (speedup 1.0000x reference)
import jax
import jax.numpy as jnp
from jax.experimental import pallas as pl

N_USER = 88
N_ITEM = 585
N = 673
D = 64
B = 1024
NEG = -1000000000.0


def _leaky(x, a):
    return jnp.where(x >= 0, x, a * x)


def _mm(a, b):
    return jnp.dot(a, b, preferred_element_type=jnp.float32)


def _first_kernel(inter_ref, ego_ref, adj_ref, wg_ref, bg_ref, wb_ref, bb_ref,
                  coefs_ref, sum_ref, t_ref):
    ii = jax.lax.broadcasted_iota(jnp.int32, (N, N), 0)
    jj = jax.lax.broadcasted_iota(jnp.int32, (N, N), 1)
    eye = (ii == jj).astype(jnp.float32)
    mt = inter_ref[...] + eye
    region = (ii < 2 * N_USER) & (jj < 2 * N_USER)
    mt = jnp.where(region & (mt > 0), 1.0, mt)
    bias = NEG * (1.0 - mt)
    m = jnp.max(bias, axis=1, keepdims=True)
    e = jnp.exp(bias - m)
    coefs_ref[...] = e / jnp.sum(e, axis=1, keepdims=True)

    ego = ego_ref[...]
    side = _mm(adj_ref[...], ego)
    sum_ref[...] = _mm(side, wg_ref[...]) + bg_ref[...]
    t_ref[...] = _mm(ego * side, wb_ref[...]) + bb_ref[...]


def _mid_kernel(coefs_ref, sum_prev_ref, x_ref, adj_ref,
                wg_ref, bg_ref, wb_ref, bb_ref,
                egon_ref, sum_ref, t_ref):
    bi = _mm(coefs_ref[...], x_ref[...])
    act = sum_prev_ref[...] + bi
    ego = _leaky(act, 0.2)
    nrm = jnp.maximum(
        jnp.sqrt(jnp.sum(ego * ego, axis=1, keepdims=True)), 1e-12)
    egon_ref[...] = ego / nrm
    side = _mm(adj_ref[...], ego)
    sum_ref[...] = _mm(side, wg_ref[...]) + bg_ref[...]
    t_ref[...] = _mm(ego * side, wb_ref[...]) + bb_ref[...]


def _head_kernel(coefs_ref, sum_prev_ref, x_ref,
                 e0_ref, e1_ref, e2_ref,
                 users_ref, pos_ref, neg_ref,
                 c1u_ref, c1i_ref, c1b_ref, c2_ref, c2b_ref, c3_ref, c3b_ref,
                 pos_out_ref, neg_out_ref):
    bi = _mm(coefs_ref[...], x_ref[...])
    act = sum_prev_ref[...] + bi
    ego = _leaky(act, 0.2)
    nrm = jnp.maximum(
        jnp.sqrt(jnp.sum(ego * ego, axis=1, keepdims=True)), 1e-12)
    e3 = ego / nrm

    iu = jax.lax.broadcasted_iota(jnp.int32, (B, N_USER), 1)
    oh_u = (users_ref[...] == iu).astype(jnp.float32)
    ii = jax.lax.broadcasted_iota(jnp.int32, (B, N_ITEM), 1)
    oh_p = (pos_ref[...] == ii).astype(jnp.float32)
    oh_n = (neg_ref[...] == ii).astype(jnp.float32)

    blocks = (e0_ref[...], e1_ref[...], e2_ref[...], e3)
    u_parts = [_mm(oh_u, blk[:N_USER, :]) for blk in blocks]
    p_parts = [_mm(oh_p, blk[N_USER:, :]) for blk in blocks]
    n_parts = [_mm(oh_n, blk[N_USER:, :]) for blk in blocks]

    c1u = c1u_ref[...]
    c1i = c1i_ref[...]

    def head(i_parts):
        h = c1b_ref[...]
        mf = jnp.zeros((B, 1), jnp.float32)
        for j in range(4):
            h = h + _mm(u_parts[j], c1u[j * D:(j + 1) * D, :])
            h = h + _mm(i_parts[j], c1i[j * D:(j + 1) * D, :])
            mf = mf + jnp.sum(u_parts[j] * i_parts[j], axis=1, keepdims=True)
        h = jax.nn.relu(h)
        h = jax.nn.relu(_mm(h, c2_ref[...]) + c2b_ref[...])
        h = jax.nn.relu(_mm(h, c3_ref[...]) + c3b_ref[...])
        return jnp.sum(0.001 * h, axis=1, keepdims=True) + mf

    pos_out_ref[...] = head(p_parts)
    neg_out_ref[...] = head(n_parts)


def _call(f, out_shapes, *args):
    return pl.pallas_call(f, out_shape=out_shapes)(*args)


def kernel(users, pos_items, neg_items, norm_adj, interaction, user_emb,
           item_emb, W_gc, b_gc, W_bi, b_bi, conv1_w, conv1_b,
           c1_w, c1_b, c2_w, c2_b, c3_w, c3_b):
    ego0 = jnp.concatenate([user_emb, item_emb], axis=0)

    f32 = jnp.float32
    nd = jax.ShapeDtypeStruct((N, D), f32)

    coefs, sum0, t0 = _call(
        _first_kernel,
        (jax.ShapeDtypeStruct((N, N), f32), nd, nd),
        interaction, ego0, norm_adj, W_gc[0], b_gc[0], W_bi[0], b_bi[0])

    x0 = t0.reshape(D, N).T
    e1n, sum1, t1 = _call(
        _mid_kernel, (nd, nd, nd),
        coefs, sum0, x0, norm_adj, W_gc[1], b_gc[1], W_bi[1], b_bi[1])

    x1 = t1.reshape(D, N).T
    e2n, sum2, t2 = _call(
        _mid_kernel, (nd, nd, nd),
        coefs, sum1, x1, norm_adj, W_gc[2], b_gc[2], W_bi[2], b_bi[2])

    x2 = t2.reshape(D, N).T
    pos_out, neg_out = _call(
        _head_kernel,
        (jax.ShapeDtypeStruct((B, 1), f32), jax.ShapeDtypeStruct((B, 1), f32)),
        coefs, sum2, x2,
        ego0, e1n, e2n,
        users.reshape(B, 1).astype(jnp.int32),
        pos_items.reshape(B, 1).astype(jnp.int32),
        neg_items.reshape(B, 1).astype(jnp.int32),
        c1_w.T[:256, :], c1_w.T[256:, :], c1_b.reshape(1, 256),
        c2_w.T, c2_b.reshape(1, 256), c3_w.T, c3_b.reshape(1, 256))

    return (pos_out.reshape(B), neg_out.reshape(B))


# speedup vs baseline: 1.0453x; 1.0453x over previous
import jax
import jax.numpy as jnp
from jax.experimental import pallas as pl
from jax.experimental.pallas import tpu as pltpu

N_USER = 88
N_ITEM = 585
N = 673
D = 64
B = 1024
L = 3
NA = 11
NPAD = NA * D
NROW = 680
NEG = -1000000000.0


def _leaky(x, a):
    return jnp.where(x >= 0, x, a * x)


def _mm(a, b):
    return jnp.dot(a, b, preferred_element_type=jnp.float32)


def _mm_t(a, b):
    return jax.lax.dot_general(a, b, (((1,), (1,)), ((), ())),
                               preferred_element_type=jnp.float32)


def _shuffle(t, tpad_ref, seq_ref):
    tpad_ref[0:N, :] = t
    tpad_ref[N:NROW, :] = jnp.zeros((NROW - N, D), jnp.float32)
    bb = jax.lax.broadcasted_iota(jnp.int32, (NA, D), 1)
    for d in range(D):
        q, s = (N * d) // D, (N * d) % D
        w = tpad_ref[q:q + NA + 1, :]
        if s == 0:
            seq_ref[d] = w[0:NA, :]
        else:
            lo = jnp.roll(w[0:NA, :], -s, axis=1)
            hi = jnp.roll(w[1:NA + 1, :], -s, axis=1)
            seq_ref[d] = jnp.where(bb < D - s, lo, hi)


def _mono_kernel(inter3_ref, ego_ref, adj_ref,
                 wg0_ref, wg1_ref, wg2_ref, bg0_ref, bg1_ref, bg2_ref,
                 wb0_ref, wb1_ref, wb2_ref, bb0_ref, bb1_ref, bb2_ref,
                 users_ref, pos_ref, neg_ref,
                 c1u_ref, c1i_ref, c1b_ref, c2_ref, c2b_ref, c3_ref, c3b_ref,
                 pos_out_ref, neg_out_ref,
                 tpad_ref, seq_ref):
    i0 = jax.lax.broadcasted_iota(jnp.int32, (N, NA, D), 0)
    ia = jax.lax.broadcasted_iota(jnp.int32, (N, NA, D), 1)
    ib = jax.lax.broadcasted_iota(jnp.int32, (N, NA, D), 2)
    col = ia * D + ib
    eye3 = (i0 == col).astype(jnp.float32)
    mt = inter3_ref[...] + eye3
    region = (i0 < 2 * N_USER) & (col < 2 * N_USER)
    mt = jnp.where(region & (mt > 0), 1.0, mt)
    e3 = jnp.exp(NEG * (1.0 - mt))
    ssum = jnp.sum(jnp.sum(e3, axis=2, keepdims=True), axis=1, keepdims=True)
    coefs3 = e3 / ssum

    adj = adj_ref[...]
    ego = ego_ref[...]
    wgs = (wg0_ref, wg1_ref, wg2_ref)
    bgs = (bg0_ref, bg1_ref, bg2_ref)
    wbs = (wb0_ref, wb1_ref, wb2_ref)
    bbs = (bb0_ref, bb1_ref, bb2_ref)
    alls = [ego]
    for k in range(L):
        side = _mm(adj, ego)
        sum_emb = _mm(side, wgs[k][...]) + bgs[k][...]
        t = _mm(ego * side, wbs[k][...]) + bbs[k][...]
        _shuffle(t, tpad_ref, seq_ref)
        bi = _mm_t(coefs3[:, 0, :], seq_ref[:, 0, :])
        for a in range(1, NA):
            bi = bi + _mm_t(coefs3[:, a, :], seq_ref[:, a, :])
        ego = _leaky(sum_emb + bi, 0.2)
        nrm = jnp.maximum(
            jnp.sqrt(jnp.sum(ego * ego, axis=1, keepdims=True)), 1e-12)
        alls.append(ego / nrm)

    iu = jax.lax.broadcasted_iota(jnp.int32, (B, N_USER), 1)
    oh_u = (users_ref[...] == iu).astype(jnp.float32)
    it = jax.lax.broadcasted_iota(jnp.int32, (B, N_ITEM), 1)
    oh_p = (pos_ref[...] == it).astype(jnp.float32)
    oh_n = (neg_ref[...] == it).astype(jnp.float32)

    u_parts = [_mm(oh_u, blk[:N_USER, :]) for blk in alls]
    p_parts = [_mm(oh_p, blk[N_USER:, :]) for blk in alls]
    n_parts = [_mm(oh_n, blk[N_USER:, :]) for blk in alls]

    c1u = c1u_ref[...]
    c1i = c1i_ref[...]

    u_h = c1b_ref[...]
    for j in range(4):
        u_h = u_h + _mm(u_parts[j], c1u[j * D:(j + 1) * D, :])

    def head(i_parts):
        h = u_h
        mf = jnp.zeros((B, 1), jnp.float32)
        for j in range(4):
            h = h + _mm(i_parts[j], c1i[j * D:(j + 1) * D, :])
            mf = mf + jnp.sum(u_parts[j] * i_parts[j], axis=1, keepdims=True)
        h = jax.nn.relu(h)
        h = jax.nn.relu(_mm(h, c2_ref[...]) + c2b_ref[...])
        h = jax.nn.relu(_mm(h, c3_ref[...]) + c3b_ref[...])
        return jnp.sum(0.001 * h, axis=1, keepdims=True) + mf

    pos_out_ref[...] = head(p_parts)
    neg_out_ref[...] = head(n_parts)


def kernel(users, pos_items, neg_items, norm_adj, interaction, user_emb,
           item_emb, W_gc, b_gc, W_bi, b_bi, conv1_w, conv1_b,
           c1_w, c1_b, c2_w, c2_b, c3_w, c3_b):
    ego0 = jnp.concatenate([user_emb, item_emb], axis=0)
    inter3 = jnp.pad(interaction, ((0, 0), (0, NPAD - N))).reshape(N, NA, D)
    f32 = jnp.float32

    pos_out, neg_out = pl.pallas_call(
        _mono_kernel,
        out_shape=(jax.ShapeDtypeStruct((B, 1), f32),
                   jax.ShapeDtypeStruct((B, 1), f32)),
        scratch_shapes=[
            pltpu.VMEM((NROW, D), f32),
            pltpu.VMEM((D, NA, D), f32),
        ],
    )(
        inter3, ego0, norm_adj,
        W_gc[0], W_gc[1], W_gc[2], b_gc[0], b_gc[1], b_gc[2],
        W_bi[0], W_bi[1], W_bi[2], b_bi[0], b_bi[1], b_bi[2],
        users.reshape(B, 1).astype(jnp.int32),
        pos_items.reshape(B, 1).astype(jnp.int32),
        neg_items.reshape(B, 1).astype(jnp.int32),
        c1_w.T[:256, :], c1_w.T[256:, :], c1_b.reshape(1, 256),
        c2_w.T, c2_b.reshape(1, 256), c3_w.T, c3_b.reshape(1, 256))

    return (pos_out.reshape(B), neg_out.reshape(B))


# speedup vs baseline: 1.4592x; 1.3959x over previous
import jax
import jax.numpy as jnp
from jax.experimental import pallas as pl
from jax.experimental.pallas import tpu as pltpu

N_USER = 88
N_ITEM = 585
N = 673
D = 64
B = 1024
L = 3
NA = 11
NROW = 680
NEG = -1000000000.0


def _leaky(x, a):
    return jnp.where(x >= 0, x, a * x)


def _mm(a, b):
    return jnp.dot(a, b, preferred_element_type=jnp.float32)


def _mm_t(a, b):
    return jax.lax.dot_general(a, b, (((1,), (1,)), ((), ())),
                               preferred_element_type=jnp.float32)


def _shuffle(t, tpad_ref, seq_ref):
    tpad_ref[0:N, :] = t
    tpad_ref[N:NROW, :] = jnp.zeros((NROW - N, D), jnp.float32)
    bb = jax.lax.broadcasted_iota(jnp.int32, (NA, D), 1)
    for d in range(D):
        q, s = (N * d) // D, (N * d) % D
        w = tpad_ref[q:q + NA + 1, :]
        if s == 0:
            seq_ref[d] = w[0:NA, :]
        else:
            lo = jnp.roll(w[0:NA, :], -s, axis=1)
            hi = jnp.roll(w[1:NA + 1, :], -s, axis=1)
            seq_ref[d] = jnp.where(bb < D - s, lo, hi)


def _mono_kernel(inter_ref, ue_ref, ie_ref, adj_ref,
                 wg_ref, bg_ref, wb_ref, bb_ref,
                 users_ref, pos_ref, neg_ref,
                 c1_ref, c1b_ref, c2_ref, c2b_ref, c3_ref, c3b_ref,
                 pos_out_ref, neg_out_ref,
                 tpad_ref, seq_ref):
    ii = jax.lax.broadcasted_iota(jnp.int32, (N, N), 0)
    jj = jax.lax.broadcasted_iota(jnp.int32, (N, N), 1)
    eye = (ii == jj).astype(jnp.float32)
    mt = inter_ref[...] + eye
    region = (ii < 2 * N_USER) & (jj < 2 * N_USER)
    mt = jnp.where(region & (mt > 0), 1.0, mt)
    e2 = jnp.exp(NEG * (1.0 - mt))
    coefs = e2 / jnp.sum(e2, axis=1, keepdims=True)

    adj = adj_ref[...]
    ego = jnp.concatenate([ue_ref[...], ie_ref[...]], axis=0)
    alls = [ego]
    for k in range(L):
        side = _mm(adj, ego)
        sum_emb = _mm(side, wg_ref[k]) + bg_ref[k]
        t = _mm(ego * side, wb_ref[k]) + bb_ref[k]
        _shuffle(t, tpad_ref, seq_ref)
        bi = _mm_t(coefs[:, 0:D], seq_ref[:, 0, :])
        for a in range(1, NA - 1):
            bi = bi + _mm_t(coefs[:, a * D:(a + 1) * D], seq_ref[:, a, :])
        bi = bi + _mm_t(coefs[:, (NA - 1) * D:N], seq_ref[:, NA - 1, 0:N - (NA - 1) * D])
        ego = _leaky(sum_emb + bi, 0.2)
        nrm = jnp.maximum(
            jnp.sqrt(jnp.sum(ego * ego, axis=1, keepdims=True)), 1e-12)
        alls.append(ego / nrm)

    iu = jax.lax.broadcasted_iota(jnp.int32, (B, N_USER), 1)
    oh_u = (users_ref[...] == iu).astype(jnp.float32)
    it = jax.lax.broadcasted_iota(jnp.int32, (B, N_ITEM), 1)
    oh_p = (pos_ref[...] == it).astype(jnp.float32)
    oh_n = (neg_ref[...] == it).astype(jnp.float32)

    u_parts = [_mm(oh_u, blk[:N_USER, :]) for blk in alls]
    p_parts = [_mm(oh_p, blk[N_USER:, :]) for blk in alls]
    n_parts = [_mm(oh_n, blk[N_USER:, :]) for blk in alls]

    c1 = c1_ref[...]

    u_h = c1b_ref[...]
    for j in range(4):
        u_h = u_h + _mm_t(u_parts[j], c1[:, j * D:(j + 1) * D])

    def head(i_parts):
        h = u_h
        mf = jnp.zeros((B, 1), jnp.float32)
        for j in range(4):
            h = h + _mm_t(i_parts[j], c1[:, 256 + j * D:256 + (j + 1) * D])
            mf = mf + jnp.sum(u_parts[j] * i_parts[j], axis=1, keepdims=True)
        h = jax.nn.relu(h)
        h = jax.nn.relu(_mm_t(h, c2_ref[...]) + c2b_ref[...])
        h = jax.nn.relu(_mm_t(h, c3_ref[...]) + c3b_ref[...])
        return jnp.sum(0.001 * h, axis=1, keepdims=True) + mf

    pos_out_ref[...] = head(p_parts)
    neg_out_ref[...] = head(n_parts)


def kernel(users, pos_items, neg_items, norm_adj, interaction, user_emb,
           item_emb, W_gc, b_gc, W_bi, b_bi, conv1_w, conv1_b,
           c1_w, c1_b, c2_w, c2_b, c3_w, c3_b):
    f32 = jnp.float32

    pos_out, neg_out = pl.pallas_call(
        _mono_kernel,
        out_shape=(jax.ShapeDtypeStruct((B, 1), f32),
                   jax.ShapeDtypeStruct((B, 1), f32)),
        scratch_shapes=[
            pltpu.VMEM((NROW, D), f32),
            pltpu.VMEM((D, NA, D), f32),
        ],
    )(
        interaction, user_emb, item_emb, norm_adj,
        W_gc, b_gc, W_bi, b_bi,
        users.reshape(B, 1).astype(jnp.int32),
        pos_items.reshape(B, 1).astype(jnp.int32),
        neg_items.reshape(B, 1).astype(jnp.int32),
        c1_w, c1_b.reshape(1, 256),
        c2_w, c2_b.reshape(1, 256), c3_w, c3_b.reshape(1, 256))

    return (pos_out.reshape(B), neg_out.reshape(B))


# speedup vs baseline: 2.3592x; 1.6168x over previous
import jax
import jax.numpy as jnp
from jax.experimental import pallas as pl
from jax.experimental.pallas import tpu as pltpu

N_USER = 88
N_ITEM = 585
N = 673
D = 64
B = 1024
L = 3
NA = 11
NROW = 680
NEG = -1000000000.0


def _leaky(x, a):
    return jnp.where(x >= 0, x, a * x)


def _mm(a, b):
    return jnp.dot(a, b, preferred_element_type=jnp.float32)


def _mm_t(a, b):
    return jax.lax.dot_general(a, b, (((1,), (1,)), ((), ())),
                               preferred_element_type=jnp.float32)


def _shuffle(t, tpad_ref, seq_ref):
    tpad_ref[0:N, :] = t
    tpad_ref[N:NROW, :] = jnp.zeros((NROW - N, D), jnp.float32)
    bb = jax.lax.broadcasted_iota(jnp.int32, (NA, D), 1)
    for d in range(D):
        q, s = (N * d) // D, (N * d) % D
        w = tpad_ref[q:q + NA + 1, :]
        if s == 0:
            seq_ref[d] = w[0:NA, :]
        else:
            lo = jnp.roll(w[0:NA, :], -s, axis=1)
            hi = jnp.roll(w[1:NA + 1, :], -s, axis=1)
            seq_ref[d] = jnp.where(bb < D - s, lo, hi)


def _mono_kernel(inter_ref, uet_ref, iet_ref, adj_ref,
                 wg_ref, bg_ref, wb_ref, bb_ref,
                 users_ref, pos_ref, neg_ref,
                 c1_ref, c1b_ref, c2_ref, c2b_ref, c3_ref, c3b_ref,
                 pos_out_ref, neg_out_ref,
                 tpad_ref, seq_ref):
    ii = jax.lax.broadcasted_iota(jnp.int32, (N, N), 0)
    jj = jax.lax.broadcasted_iota(jnp.int32, (N, N), 1)
    eye = (ii == jj).astype(jnp.float32)
    mt = inter_ref[...] + eye
    region = (ii < 2 * N_USER) & (jj < 2 * N_USER)
    mt = jnp.where(region & (mt > 0), 1.0, mt)
    e2 = jnp.exp(NEG * (1.0 - mt))
    coefs = e2 / jnp.sum(e2, axis=1, keepdims=True)

    adj = adj_ref[...]
    ego = jnp.concatenate(
        [jnp.transpose(uet_ref[...]), jnp.transpose(iet_ref[...])], axis=0)
    alls = [ego]
    for k in range(L):
        side = _mm(adj, ego)
        sum_emb = _mm(side, wg_ref[k]) + bg_ref[k]
        t = _mm(ego * side, wb_ref[k]) + bb_ref[k]
        _shuffle(t, tpad_ref, seq_ref)
        bi = _mm_t(coefs[:, 0:D], seq_ref[:, 0, :])
        for a in range(1, NA - 1):
            bi = bi + _mm_t(coefs[:, a * D:(a + 1) * D], seq_ref[:, a, :])
        bi = bi + _mm_t(coefs[:, (NA - 1) * D:N], seq_ref[:, NA - 1, 0:N - (NA - 1) * D])
        ego = _leaky(sum_emb + bi, 0.2)
        nrm = jnp.maximum(
            jnp.sqrt(jnp.sum(ego * ego, axis=1, keepdims=True)), 1e-12)
        alls.append(ego / nrm)

    def _lane_row(idx_ref):
        return jnp.concatenate(
            [idx_ref[r:r + 1, :] for r in range(8)], axis=1)

    iu = jax.lax.broadcasted_iota(jnp.int32, (N_USER, B), 0)
    oh_u = (_lane_row(users_ref) == iu).astype(jnp.float32)
    it = jax.lax.broadcasted_iota(jnp.int32, (N_ITEM, B), 0)
    oh_p = (_lane_row(pos_ref) == it).astype(jnp.float32)
    oh_n = (_lane_row(neg_ref) == it).astype(jnp.float32)

    def _gsel(oh, blk):
        return jax.lax.dot_general(oh, blk, (((0,), (0,)), ((), ())),
                                   preferred_element_type=jnp.float32)

    u_parts = [_gsel(oh_u, blk[:N_USER, :]) for blk in alls]
    p_parts = [_gsel(oh_p, blk[N_USER:, :]) for blk in alls]
    n_parts = [_gsel(oh_n, blk[N_USER:, :]) for blk in alls]

    c1 = c1_ref[...]

    u_h = c1b_ref[...]
    for j in range(4):
        u_h = u_h + _mm_t(u_parts[j], c1[:, j * D:(j + 1) * D])

    def head(i_parts):
        h = u_h
        mf = jnp.zeros((B, 1), jnp.float32)
        for j in range(4):
            h = h + _mm_t(i_parts[j], c1[:, 256 + j * D:256 + (j + 1) * D])
            mf = mf + jnp.sum(u_parts[j] * i_parts[j], axis=1, keepdims=True)
        h = jax.nn.relu(h)
        h = jax.nn.relu(_mm_t(h, c2_ref[...]) + c2b_ref[...])
        h = jax.nn.relu(_mm_t(h, c3_ref[...]) + c3b_ref[...])
        return jnp.sum(0.001 * h, axis=1, keepdims=True) + mf

    ieye = jax.lax.broadcasted_iota(jnp.int32, (128, 128), 0)
    jeye = jax.lax.broadcasted_iota(jnp.int32, (128, 128), 1)
    eye128 = (ieye == jeye).astype(jnp.float32)

    def _to_grid(col):
        rows = [jax.lax.dot_general(col[128 * r:128 * (r + 1), :], eye128,
                                    (((0,), (0,)), ((), ())),
                                    preferred_element_type=jnp.float32)
                for r in range(8)]
        return jnp.concatenate(rows, axis=0)

    pos_out_ref[...] = _to_grid(head(p_parts))
    neg_out_ref[...] = _to_grid(head(n_parts))


def kernel(users, pos_items, neg_items, norm_adj, interaction, user_emb,
           item_emb, W_gc, b_gc, W_bi, b_bi, conv1_w, conv1_b,
           c1_w, c1_b, c2_w, c2_b, c3_w, c3_b):
    f32 = jnp.float32

    pos_out, neg_out = pl.pallas_call(
        _mono_kernel,
        out_shape=(jax.ShapeDtypeStruct((8, 128), f32),
                   jax.ShapeDtypeStruct((8, 128), f32)),
        scratch_shapes=[
            pltpu.VMEM((NROW, D), f32),
            pltpu.VMEM((D, NA, D), f32),
        ],
    )(
        interaction, user_emb.T, item_emb.T, norm_adj,
        W_gc, b_gc, W_bi, b_bi,
        users.astype(jnp.int32).reshape(8, 128),
        pos_items.astype(jnp.int32).reshape(8, 128),
        neg_items.astype(jnp.int32).reshape(8, 128),
        c1_w, c1_b.reshape(1, 256),
        c2_w, c2_b.reshape(1, 256), c3_w, c3_b.reshape(1, 256))

    return (pos_out.reshape(B), neg_out.reshape(B))
